# baseline (speedup 1.0000x reference)
import functools

import jax
import jax.numpy as jnp
from jax.experimental import pallas as pl
from jax.experimental.pallas import tpu as pltpu

_BN = 1024
_NCHUNK = 4
_BNORM = 2048


def _norm_body(x_ref, o_ref):
    x = x_ref[...]
    n = jnp.sqrt(jnp.sum(x * x, axis=1, keepdims=True))
    o_ref[...] = (x / (n + 1e-12)).astype(jnp.bfloat16)


def _normalize_bf16(a):
    rows, d = a.shape
    bn = min(_BNORM, rows)
    return pl.pallas_call(
        _norm_body,
        grid=(rows // bn,),
        in_specs=[pl.BlockSpec((bn, d), lambda i: (i, 0))],
        out_specs=pl.BlockSpec((bn, d), lambda i: (i, 0)),
        out_shape=jax.ShapeDtypeStruct((rows, d), jnp.bfloat16),
    )(a)


def _main_body(lab_ref, x_ref, cn_ref, out_ref, *, n_classes, nchunk):
    x = x_ref[...]
    s2 = jnp.sum(x * x, axis=1, keepdims=True)
    r = 1.0 / (jnp.sqrt(s2) + 1e-12)
    xn = (x * r).astype(jnp.bfloat16)

    k = cn_ref.shape[0]
    ck = k // nchunk
    for s in range(nchunk):
        sim = jax.lax.dot_general(
            xn, cn_ref[s * ck:(s + 1) * ck, :], (((1,), (1,)), ((), ())),
            preferred_element_type=jnp.float32)
        e = (jnp.exp(sim) * 46.0 + 0.5).astype(jnp.int8)
        lab = lab_ref[0, :, s * ck:(s + 1) * ck]
        oh_t = (jax.lax.broadcasted_iota(jnp.int32, (n_classes, ck), 0)
                == lab).astype(jnp.int8)
        part = jax.lax.dot_general(
            e, oh_t, (((1,), (1,)), ((), ())),
            preferred_element_type=jnp.int32)
        if s == 0:
            out_ref[...] = part.astype(jnp.float32)
        else:
            out_ref[...] += part.astype(jnp.float32)

    res = out_ref[...]
    out_ref[...] = res / jnp.sum(res, axis=1, keepdims=True)


def kernel(x, centroids, centroid_labels):
    n, d = x.shape
    k = centroids.shape[0]
    labels = centroid_labels.astype(jnp.int32)
    n_classes = 1024

    cn = _normalize_bf16(centroids)

    bn = min(_BN, n)
    lab3 = labels.reshape(1, 1, k)

    body = functools.partial(_main_body, n_classes=n_classes,
                             nchunk=_NCHUNK)
    return pl.pallas_call(
        body,
        grid=(n // bn,),
        in_specs=[
            pl.BlockSpec((1, 1, k), lambda i: (0, 0, 0)),
            pl.BlockSpec((bn, d), lambda i: (i, 0)),
            pl.BlockSpec((k, d), lambda i: (0, 0)),
        ],
        out_specs=pl.BlockSpec((bn, n_classes), lambda i: (i, 0)),
        out_shape=jax.ShapeDtypeStruct((n, n_classes), jnp.float32),
        compiler_params=pltpu.CompilerParams(
            dimension_semantics=("arbitrary",)),
    )(lab3, x, cn)


# speedup vs baseline: 1.0086x; 1.0086x over previous
import functools

import jax
import jax.numpy as jnp
from jax.experimental import pallas as pl
from jax.experimental.pallas import tpu as pltpu

_BN = 1024
_NCHUNK = 4
_BNORM = 2048


def _norm_body(x_ref, o_ref):
    x = x_ref[...]
    n = jnp.sqrt(jnp.sum(x * x, axis=1, keepdims=True))
    o_ref[...] = (x / (n + 1e-12)).astype(jnp.bfloat16)


def _normalize_bf16(a):
    rows, d = a.shape
    bn = min(_BNORM, rows)
    return pl.pallas_call(
        _norm_body,
        grid=(rows // bn,),
        in_specs=[pl.BlockSpec((bn, d), lambda i: (i, 0))],
        out_specs=pl.BlockSpec((bn, d), lambda i: (i, 0)),
        out_shape=jax.ShapeDtypeStruct((rows, d), jnp.bfloat16),
    )(a)


def _main_body(lab_ref, x_ref, cn_ref, out_ref, *, n_classes, nchunk):
    x = x_ref[...]
    s2 = jnp.sum(x * x, axis=1, keepdims=True)
    r = 1.0 / (jnp.sqrt(s2) + 1e-12)
    xn = (x * r).astype(jnp.bfloat16)

    k = cn_ref.shape[0]
    ck = k // nchunk
    for s in range(nchunk):
        sim = jax.lax.dot_general(
            xn, cn_ref[s * ck:(s + 1) * ck, :], (((1,), (1,)), ((), ())),
            preferred_element_type=jnp.float32)
        e = jnp.exp(sim).astype(jnp.bfloat16)
        lab = lab_ref[0, :, s * ck:(s + 1) * ck]
        oh_t = (jax.lax.broadcasted_iota(jnp.int32, (n_classes, ck), 0)
                == lab).astype(jnp.bfloat16)
        part = jax.lax.dot_general(
            e, oh_t, (((1,), (1,)), ((), ())),
            preferred_element_type=jnp.float32)
        if s == 0:
            out_ref[...] = part
        else:
            out_ref[...] += part

    res = out_ref[...]
    out_ref[...] = res / jnp.sum(res, axis=1, keepdims=True)


def kernel(x, centroids, centroid_labels):
    n, d = x.shape
    k = centroids.shape[0]
    labels = centroid_labels.astype(jnp.int32)
    n_classes = 1024

    cn = _normalize_bf16(centroids)

    bn = min(_BN, n)
    lab3 = labels.reshape(1, 1, k)

    body = functools.partial(_main_body, n_classes=n_classes,
                             nchunk=_NCHUNK)
    return pl.pallas_call(
        body,
        grid=(n // bn,),
        in_specs=[
            pl.BlockSpec((1, 1, k), lambda i: (0, 0, 0)),
            pl.BlockSpec((bn, d), lambda i: (i, 0)),
            pl.BlockSpec((k, d), lambda i: (0, 0)),
        ],
        out_specs=pl.BlockSpec((bn, n_classes), lambda i: (i, 0)),
        out_shape=jax.ShapeDtypeStruct((n, n_classes), jnp.float32),
        compiler_params=pltpu.CompilerParams(
            dimension_semantics=("arbitrary",)),
    )(lab3, x, cn)
